# baseline (speedup 1.0000x reference)
import jax
import jax.numpy as jnp
from jax.experimental import pallas as pl
from jax.experimental.pallas import tpu as pltpu

INPUT_SIZE = 256
OUT_SIZE = 256
MEMORY_FEATURE = 128

_BLOCK_M = 1024


def _body(x_ref, memslice_ref, memany_ref, w_ref, b_ref,
          out_ref, mstate_ref, copy_sem):
    i = pl.program_id(0)
    nsteps = pl.num_programs(0)

    @pl.when(i == 0)
    def _start_copy():
        pltpu.make_async_copy(memany_ref, mstate_ref, copy_sem).start()

    acc = jnp.dot(x_ref[...], w_ref[:INPUT_SIZE, :],
                  preferred_element_type=jnp.float32)
    acc = acc + jnp.dot(memslice_ref[...], w_ref[INPUT_SIZE:, :],
                        preferred_element_type=jnp.float32)
    out_ref[...] = acc + b_ref[...]

    @pl.when(i == nsteps - 1)
    def _finish_copy():
        pltpu.make_async_copy(memany_ref, mstate_ref, copy_sem).wait()


def kernel(x, mem, W, b):
    batch, _ = x.shape
    nblocks = batch // _BLOCK_M
    b2 = b.reshape(1, OUT_SIZE)
    out, mem_state = pl.pallas_call(
        _body,
        grid=(nblocks,),
        in_specs=[
            pl.BlockSpec((_BLOCK_M, INPUT_SIZE), lambda i: (i, 0)),
            pl.BlockSpec((_BLOCK_M, MEMORY_FEATURE), lambda i: (i, 0)),
            pl.BlockSpec(memory_space=pltpu.MemorySpace.HBM),
            pl.BlockSpec((INPUT_SIZE + MEMORY_FEATURE, OUT_SIZE),
                         lambda i: (0, 0)),
            pl.BlockSpec((1, OUT_SIZE), lambda i: (0, 0)),
        ],
        out_specs=[
            pl.BlockSpec((_BLOCK_M, OUT_SIZE), lambda i: (i, 0)),
            pl.BlockSpec(memory_space=pltpu.MemorySpace.HBM),
        ],
        out_shape=[
            jax.ShapeDtypeStruct((batch, OUT_SIZE), jnp.float32),
            jax.ShapeDtypeStruct(mem.shape, mem.dtype),
        ],
        scratch_shapes=[pltpu.SemaphoreType.DMA],
    )(x, mem, mem, W, b2)
    return (out, mem_state)


# speedup vs baseline: 27.2068x; 27.2068x over previous
import jax
import jax.numpy as jnp
from jax.experimental import pallas as pl
from jax.experimental.pallas import tpu as pltpu

INPUT_SIZE = 256
OUT_SIZE = 256
MEMORY_FEATURE = 128

_STEPS = 32


def _body(x_ref, memslice_ref, memcopy_ref, w_ref, b_ref, out_ref, mstate_ref):
    mstate_ref[...] = memcopy_ref[...]
    acc = jnp.dot(x_ref[...], w_ref[:INPUT_SIZE, :],
                  preferred_element_type=jnp.float32)
    acc = acc + jnp.dot(memslice_ref[...], w_ref[INPUT_SIZE:, :],
                        preferred_element_type=jnp.float32)
    out_ref[...] = acc + b_ref[...]


def kernel(x, mem, W, b):
    batch, _ = x.shape
    memory_size = mem.shape[0]
    bm = batch // _STEPS
    cm = memory_size // _STEPS
    b2 = b.reshape(1, OUT_SIZE)
    out, mem_state = pl.pallas_call(
        _body,
        grid=(_STEPS,),
        in_specs=[
            pl.BlockSpec((bm, INPUT_SIZE), lambda i: (i, 0)),
            pl.BlockSpec((bm, MEMORY_FEATURE), lambda i: (i, 0)),
            pl.BlockSpec((cm, MEMORY_FEATURE), lambda i: (i, 0)),
            pl.BlockSpec((INPUT_SIZE + MEMORY_FEATURE, OUT_SIZE),
                         lambda i: (0, 0)),
            pl.BlockSpec((1, OUT_SIZE), lambda i: (0, 0)),
        ],
        out_specs=[
            pl.BlockSpec((bm, OUT_SIZE), lambda i: (i, 0)),
            pl.BlockSpec((cm, MEMORY_FEATURE), lambda i: (i, 0)),
        ],
        out_shape=[
            jax.ShapeDtypeStruct((batch, OUT_SIZE), jnp.float32),
            jax.ShapeDtypeStruct(mem.shape, mem.dtype),
        ],
    )(x, mem, mem, W, b2)
    return (out, mem_state)
